# Initial kernel scaffold; baseline (speedup 1.0000x reference)
import jax
import jax.numpy as jnp
from jax.experimental import pallas as pl


def kernel(x, W1, b1, Wa, ba, Wb, bb, Wc, bc, Wcls, bcls):
    raise NotImplementedError("write your pallas kernel here")


# speedup vs baseline: 1.2516x; 1.2516x over previous
import jax
import jax.numpy as jnp
from jax.experimental import pallas as pl
from jax.experimental.pallas import tpu as pltpu

_TN = 2000


def _fused(x_ref, w1t_ref, b1_ref, wat_ref, ba_ref, wbt_ref, bb_ref,
           wct_ref, bc_ref, wcls_ref, bcls_ref,
           attn_ref, logits_ref, yprob_ref, yhat_ref,
           m_ref, s_ref, macc_ref):
    i = pl.program_id(0)
    nt = pl.num_programs(0)

    @pl.when(i == 0)
    def _init():
        m_ref[0, 0] = -jnp.inf
        s_ref[0, 0] = 0.0
        macc_ref[...] = jnp.zeros_like(macc_ref)

    xb = x_ref[...].astype(jnp.bfloat16)
    h = jnp.dot(xb, w1t_ref[...], preferred_element_type=jnp.float32)
    h = jnp.maximum(h + b1_ref[...], 0.0)
    hb = h.astype(jnp.bfloat16)
    a = jnp.tanh(jnp.dot(hb, wat_ref[...], preferred_element_type=jnp.float32)
                 + ba_ref[...])
    g = jax.nn.sigmoid(jnp.dot(hb, wbt_ref[...], preferred_element_type=jnp.float32)
                       + bb_ref[...])
    ab = a * g
    A = jnp.dot(ab, wct_ref[...], preferred_element_type=jnp.float32) + bc_ref[...]
    attn_ref[...] = A

    m_old = m_ref[0, 0]
    m_new = jnp.maximum(m_old, jnp.max(A))
    corr = jnp.exp(m_old - m_new)
    w = jnp.exp(A - m_new)
    s_new = s_ref[0, 0] * corr + jnp.sum(w)
    macc_new = macc_ref[...] * corr + jnp.sum(w * h, axis=0, keepdims=True)
    m_ref[0, 0] = m_new
    s_ref[0, 0] = s_new
    macc_ref[...] = macc_new

    @pl.when(i == nt - 1)
    def _finalize():
        pooled = macc_new / s_new
        logits_ref[...] = (jnp.sum(pooled * wcls_ref[...], axis=1, keepdims=True)
                           + bcls_ref[...])
        yprob_ref[...] = jnp.ones((1, 1), jnp.float32)
        yhat_ref[...] = jnp.zeros((1, 1), jnp.int32)


def kernel(x, W1, b1, Wa, ba, Wb, bb, Wc, bc, Wcls, bcls):
    N, L = x.shape
    D = Wa.shape[0]
    tn = _TN if N % _TN == 0 else next(t for t in (1000, 500, 200, 100, 8, 1)
                                       if N % t == 0)

    w1t = W1.T.astype(jnp.bfloat16)
    wat = Wa.T.astype(jnp.bfloat16)
    wbt = Wb.T.astype(jnp.bfloat16)
    wct = Wc.T

    full = lambda shape: pl.BlockSpec(shape, lambda i: (0, 0))
    out = pl.pallas_call(
        _fused,
        grid=(N // tn,),
        in_specs=[
            pl.BlockSpec((tn, L), lambda i: (i, 0)),
            full((L, L)), full((1, L)),
            full((L, D)), full((1, D)),
            full((L, D)), full((1, D)),
            full((D, 1)), full((1, 1)),
            full((1, L)), full((1, 1)),
        ],
        out_specs=[
            pl.BlockSpec((tn, 1), lambda i: (i, 0)),
            full((1, 1)), full((1, 1)), full((1, 1)),
        ],
        out_shape=[
            jax.ShapeDtypeStruct((N, 1), jnp.float32),
            jax.ShapeDtypeStruct((1, 1), jnp.float32),
            jax.ShapeDtypeStruct((1, 1), jnp.float32),
            jax.ShapeDtypeStruct((1, 1), jnp.int32),
        ],
        scratch_shapes=[
            pltpu.SMEM((1, 1), jnp.float32),
            pltpu.SMEM((1, 1), jnp.float32),
            pltpu.VMEM((1, L), jnp.float32),
        ],
    )(x, w1t, b1.reshape(1, L), wat, ba.reshape(1, D),
      wbt, bb.reshape(1, D), wct, bc.reshape(1, 1),
      Wcls, bcls.reshape(1, 1))

    attn_col, logits, y_prob, y_hat = out
    return (logits, y_prob, y_hat, attn_col.reshape(1, N))
